# Initial kernel scaffold; baseline (speedup 1.0000x reference)
import functools

import jax
import jax.numpy as jnp
from jax import lax
from jax.experimental import pallas as pl
from jax.experimental.pallas import tpu as pltpu
from jax.experimental.pallas import tpu_sc as plsc

PAD = 0
LANES = 16


@functools.partial(jax.jit, static_argnames=("b_per_w", "chunk"))
def _emb_lookup(table, idx, *, b_per_w, chunk):
    vocab, dim = table.shape
    (batch,) = idx.shape
    n_chunks = b_per_w // chunk
    col_groups = dim // LANES

    mesh = plsc.VectorSubcoreMesh(core_axis_name="c", subcore_axis_name="s")

    @functools.partial(
        pl.kernel,
        mesh=mesh,
        out_type=jax.ShapeDtypeStruct((batch, dim), jnp.float32),
        scratch_types=[
            pltpu.VMEM((b_per_w,), jnp.int32),
            pltpu.VMEM((chunk, dim), jnp.float32),
            pltpu.SemaphoreType.DMA,
        ],
    )
    def kern(table_hbm, idx_hbm, out_hbm, idx_v, rows_v, sem):
        num_cores = lax.axis_size("c")
        wid = lax.axis_index("s") * num_cores + lax.axis_index("c")
        base = wid * b_per_w
        pltpu.sync_copy(idx_hbm.at[pl.ds(base, b_per_w)], idx_v)

        def chunk_body(g, carry):
            off = g * chunk
            pltpu.async_copy(
                table_hbm.at[idx_v.at[pl.ds(off, chunk)]], rows_v, sem
            ).wait()
            nz = jnp.int32(0)
            for j in range(chunk // LANES):
                v = idx_v[pl.ds(off + j * LANES, LANES)]
                nz = nz + jnp.sum(jnp.where(v == PAD, 1, 0).astype(jnp.int32))

            @pl.when(nz > 0)
            def _fix():
                def row_body(r, c):
                    s = idx_v[off + r]

                    @pl.when(s == PAD)
                    def _zero_row():
                        def col_body(cc, c2):
                            rows_v[r, pl.ds(cc * LANES, LANES)] = jnp.zeros(
                                (LANES,), jnp.float32
                            )
                            return c2

                        lax.fori_loop(0, col_groups, col_body, 0)

                    return c

                lax.fori_loop(0, chunk, row_body, 0)

            pltpu.sync_copy(rows_v, out_hbm.at[pl.ds(base + off, chunk)])
            return carry

        lax.fori_loop(0, n_chunks, chunk_body, 0)

    return kern(table, idx)


def kernel(sentence, table):
    n_sent, n_tok = sentence.shape
    idx = sentence.reshape(-1)
    out = _emb_lookup(table, idx, b_per_w=(n_sent * n_tok) // 32, chunk=64)
    return out.reshape(n_sent, n_tok, table.shape[1])


# speedup vs baseline: 2.0010x; 2.0010x over previous
import functools

import jax
import jax.numpy as jnp
from jax import lax
from jax.experimental import pallas as pl
from jax.experimental.pallas import tpu as pltpu
from jax.experimental.pallas import tpu_sc as plsc

PAD = 0
LANES = 16


@functools.partial(jax.jit, static_argnames=("b_per_w", "chunk"))
def _emb_lookup(table, idx, *, b_per_w, chunk):
    vocab, dim = table.shape
    (batch,) = idx.shape
    n_chunks = b_per_w // chunk
    col_groups = dim // LANES

    mesh = plsc.VectorSubcoreMesh(core_axis_name="c", subcore_axis_name="s")

    @functools.partial(
        pl.kernel,
        mesh=mesh,
        out_type=jax.ShapeDtypeStruct((batch, dim), jnp.float32),
        scratch_types=[
            pltpu.VMEM((b_per_w,), jnp.int32),
            pltpu.VMEM((chunk, dim), jnp.float32),
            pltpu.SemaphoreType.DMA,
        ],
        compiler_params=pltpu.CompilerParams(needs_layout_passes=False),
    )
    def kern(table_hbm, idx_hbm, out_hbm, idx_v, rows_v, sem):
        num_cores = lax.axis_size("c")
        wid = lax.axis_index("s") * num_cores + lax.axis_index("c")
        base = wid * b_per_w
        pltpu.sync_copy(idx_hbm.at[pl.ds(base, b_per_w)], idx_v)

        def chunk_body(g, carry):
            off = g * chunk
            pltpu.async_copy(
                table_hbm.at[idx_v.at[pl.ds(off, chunk)]], rows_v, sem
            ).wait()
            for j in range(chunk // LANES):
                v = idx_v[pl.ds(off + j * LANES, LANES)]
                nz = plsc.all_reduce_population_count(v == PAD)[0]

                @pl.when(nz > 0)
                def _fix(v=v, j=j):
                    for r in range(LANES):

                        @pl.when(v[r] == PAD)
                        def _zero_row(r=r):
                            def col_body(cc, c2):
                                rows_v[
                                    j * LANES + r, pl.ds(cc * LANES, LANES)
                                ] = jnp.zeros((LANES,), jnp.float32)
                                return c2

                            lax.fori_loop(0, col_groups, col_body, 0)

            pltpu.sync_copy(rows_v, out_hbm.at[pl.ds(base + off, chunk)])
            return carry

        lax.fori_loop(0, n_chunks, chunk_body, 0)

    return kern(table, idx)


def kernel(sentence, table):
    n_sent, n_tok = sentence.shape
    idx = sentence.reshape(-1)
    out = _emb_lookup(table, idx, b_per_w=(n_sent * n_tok) // 32, chunk=64)
    return out.reshape(n_sent, n_tok, table.shape[1])
